# baseline (speedup 1.0000x reference)
import jax
import jax.numpy as jnp
from jax.experimental import pallas as pl
from jax.experimental.pallas import tpu as pltpu

B, N, D = 8, 2048, 32
BLK = 256


def _gcn_fused_kernel(text_ref, adj_ref, w_ref, b_ref, out_ref, hidden_ref):
    @pl.when(pl.program_id(1) == 0)
    def _():
        hidden_ref[...] = jnp.dot(
            text_ref[0], w_ref[...], preferred_element_type=jnp.float32
        )

    a = adj_ref[0]
    acc = jnp.dot(a, hidden_ref[...], preferred_element_type=jnp.float32)
    denom = jnp.sum(a, axis=1, keepdims=True) + 1.0
    out_ref[0] = acc / denom + b_ref[...]


def kernel(text, adj, W, b):
    b2d = b.reshape(1, D)
    grid = (B, N // BLK)
    return pl.pallas_call(
        _gcn_fused_kernel,
        grid=grid,
        in_specs=[
            pl.BlockSpec((1, N, D), lambda bi, i: (bi, 0, 0)),
            pl.BlockSpec((1, BLK, N), lambda bi, i: (bi, i, 0)),
            pl.BlockSpec((D, D), lambda bi, i: (0, 0)),
            pl.BlockSpec((1, D), lambda bi, i: (0, 0)),
        ],
        out_specs=pl.BlockSpec((1, BLK, D), lambda bi, i: (bi, i, 0)),
        out_shape=jax.ShapeDtypeStruct((B, N, D), jnp.float32),
        scratch_shapes=[pltpu.VMEM((N, D), jnp.float32)],
        compiler_params=pltpu.CompilerParams(
            dimension_semantics=("arbitrary", "arbitrary"),
        ),
    )(text, adj, W, b2d)


# speedup vs baseline: 1.4876x; 1.4876x over previous
import jax
import jax.numpy as jnp
from jax.experimental import pallas as pl
from jax.experimental.pallas import tpu as pltpu

B, N, D = 8, 2048, 32
BLK = 1024


def _gcn_fused_kernel(text_ref, adj_ref, w_ref, b_ref, out_ref, hidden_ref):
    @pl.when(pl.program_id(1) == 0)
    def _():
        hidden_ref[:, :D] = jnp.dot(
            text_ref[0], w_ref[...], preferred_element_type=jnp.float32
        )
        hidden_ref[:, D:] = jnp.ones((N, D), jnp.float32)

    a = adj_ref[0]
    acc = jnp.dot(a, hidden_ref[...], preferred_element_type=jnp.float32)
    denom = acc[:, D : D + 1] + 1.0
    out_ref[0] = acc[:, :D] / denom + b_ref[...]


def kernel(text, adj, W, b):
    b2d = b.reshape(1, D)
    grid = (B, N // BLK)
    return pl.pallas_call(
        _gcn_fused_kernel,
        grid=grid,
        in_specs=[
            pl.BlockSpec((1, N, D), lambda bi, i: (bi, 0, 0)),
            pl.BlockSpec((1, BLK, N), lambda bi, i: (bi, i, 0)),
            pl.BlockSpec((D, D), lambda bi, i: (0, 0)),
            pl.BlockSpec((1, D), lambda bi, i: (0, 0)),
        ],
        out_specs=pl.BlockSpec((1, BLK, D), lambda bi, i: (bi, i, 0)),
        out_shape=jax.ShapeDtypeStruct((B, N, D), jnp.float32),
        scratch_shapes=[pltpu.VMEM((N, 2 * D), jnp.float32)],
        compiler_params=pltpu.CompilerParams(
            dimension_semantics=("arbitrary", "arbitrary"),
        ),
    )(text, adj, W, b2d)


# speedup vs baseline: 1.4897x; 1.0014x over previous
import jax
import jax.numpy as jnp
from jax.experimental import pallas as pl
from jax.experimental.pallas import tpu as pltpu

B, N, D = 8, 2048, 32
BLK = 2048


def _gcn_fused_kernel(text_ref, adj_ref, w_ref, b_ref, out_ref, hidden_ref):
    @pl.when(pl.program_id(1) == 0)
    def _():
        hidden_ref[:, :D] = jnp.dot(
            text_ref[0], w_ref[...], preferred_element_type=jnp.float32
        )
        hidden_ref[:, D:] = jnp.ones((N, D), jnp.float32)

    a = adj_ref[0]
    acc = jnp.dot(a, hidden_ref[...], preferred_element_type=jnp.float32)
    denom = acc[:, D : D + 1] + 1.0
    out_ref[0] = acc[:, :D] / denom + b_ref[...]


def kernel(text, adj, W, b):
    b2d = b.reshape(1, D)
    grid = (B, N // BLK)
    return pl.pallas_call(
        _gcn_fused_kernel,
        grid=grid,
        in_specs=[
            pl.BlockSpec((1, N, D), lambda bi, i: (bi, 0, 0)),
            pl.BlockSpec((1, BLK, N), lambda bi, i: (bi, i, 0)),
            pl.BlockSpec((D, D), lambda bi, i: (0, 0)),
            pl.BlockSpec((1, D), lambda bi, i: (0, 0)),
        ],
        out_specs=pl.BlockSpec((1, BLK, D), lambda bi, i: (bi, i, 0)),
        out_shape=jax.ShapeDtypeStruct((B, N, D), jnp.float32),
        scratch_shapes=[pltpu.VMEM((N, 2 * D), jnp.float32)],
        compiler_params=pltpu.CompilerParams(
            dimension_semantics=("arbitrary", "arbitrary"),
        ),
    )(text, adj, W, b2d)
